# baseline (speedup 1.0000x reference)
import functools

import jax
import jax.numpy as jnp
from jax import lax
from jax.experimental import pallas as pl
from jax.experimental.pallas import tpu as pltpu
from jax.experimental.pallas import tpu_sc as plsc

_B, _L, _N, _C = 8, 12, 2048, 1
_DIM = 24
_ADIM = 80
_TS = 24 * 12
_DOW = 7
_TOT = _B * _L * _N
_OUT_D = 3 * _DIM + _ADIM
_NP = _B * _L
_NC = 2
_NS = 16
_NW = _NC * _NS
_TPW = _TOT // _NW
_BLK = 256
_NBLK = _TPW // _BLK


def _sc_body(inp_ref, ts_ref, dow_ref, wb_ref, adp_ref, out_ref,
             inp_v0, inp_v1, asm_v0, asm_v1, ts_v, dow_v, wb_v,
             si0, si1, sa, so0, so1):
    wid = lax.axis_index("s") * _NC + lax.axis_index("c")
    base0 = wid * _TPW
    pltpu.sync_copy(wb_ref, wb_v)
    pltpu.sync_copy(ts_ref, ts_v)
    pltpu.sync_copy(dow_ref, dow_v)

    lane = lax.iota(jnp.int32, 16)
    lane3 = lane * 3
    wds = [plsc.load_gather(wb_v, [jnp.full((16,), 1 + d, jnp.int32)])
           for d in range(_DIM)]
    bds = [plsc.load_gather(wb_v, [jnp.full((16,), 1 + _DIM + d, jnp.int32)])
           for d in range(_DIM)]

    bufs = ((inp_v0, asm_v0, si0, so0),
            (inp_v1, asm_v1, si1, so1))

    def pair_body(j, carry):
        for p in (0, 1):
            q = 1 - p
            inp_v, asm_v, s_in, s_out = bufs[p]
            inp_n, asm_n, s_in_n, s_out_n = bufs[q]
            k = 2 * j + p
            t0 = base0 + k * _BLK
            plane = t0 // _N
            n0 = lax.rem(t0, _N)

            def _drain_prev():
                pltpu.make_async_copy(
                    out_ref.at[0, :, pl.ds(0, _BLK)], asm_v, s_out).wait()

            pl.when(j >= 1)(_drain_prev)

            lrow = lax.rem(plane, _L) * _ADIM
            pltpu.async_copy(
                adp_ref.at[pl.ds(lrow, _ADIM), pl.ds(n0, _BLK)],
                asm_v.at[pl.ds(72, _ADIM), :], sa)

            pltpu.make_async_copy(
                inp_ref.at[pl.ds(0, _BLK * 3)], inp_v, s_in).wait()

            for g in range(_BLK // 16):
                base = g * 48
                feat = plsc.load_gather(inp_v, [lane3 + base])
                tsv = plsc.load_gather(inp_v, [lane3 + (base + 1)])
                dwv = plsc.load_gather(inp_v, [lane3 + (base + 2)])
                ts24 = tsv.astype(jnp.int32) * _DIM
                dw24 = dwv.astype(jnp.int32) * _DIM
                sl = pl.ds(g * 16, 16)
                for d in range(_DIM):
                    asm_v[d, sl] = feat * wds[d] + bds[d]
                for d in range(_DIM):
                    asm_v[24 + d, sl] = plsc.load_gather(ts_v, [ts24 + d])
                for d in range(_DIM):
                    asm_v[48 + d, sl] = plsc.load_gather(dow_v, [dw24 + d])

            nxt = base0 + lax.rem(k + 2, _NBLK) * _BLK
            pltpu.async_copy(inp_ref.at[pl.ds(nxt * 3, _BLK * 3)],
                             inp_v, s_in)

            pltpu.make_async_copy(
                adp_ref.at[pl.ds(0, _ADIM), pl.ds(0, _BLK)],
                asm_v.at[pl.ds(72, _ADIM), :], sa).wait()
            pltpu.async_copy(
                asm_v, out_ref.at[plane, :, pl.ds(n0, _BLK)], s_out)
        return carry

    pltpu.async_copy(inp_ref.at[pl.ds(base0 * 3, _BLK * 3)], inp_v0, si0)
    pltpu.async_copy(inp_ref.at[pl.ds((base0 + _BLK) * 3, _BLK * 3)],
                     inp_v1, si1)

    lax.fori_loop(0, _NBLK // 2, pair_body, 0)

    pltpu.make_async_copy(
        out_ref.at[0, :, pl.ds(0, _BLK)], asm_v0, so0).wait()
    pltpu.make_async_copy(
        out_ref.at[0, :, pl.ds(0, _BLK)], asm_v1, so1).wait()
    pltpu.make_async_copy(
        inp_ref.at[pl.ds(0, _BLK * 3)], inp_v0, si0).wait()
    pltpu.make_async_copy(
        inp_ref.at[pl.ds(0, _BLK * 3)], inp_v1, si1).wait()


@jax.jit
def kernel(input, W, b, ts_table, dow_table, adaptive):
    inp_flat = input.reshape(-1)
    wb = jnp.concatenate([jnp.zeros((1,), jnp.float32),
                          W.reshape(-1), b,
                          jnp.zeros((7,), jnp.float32)])
    ts1 = ts_table.reshape(-1)
    dow1 = dow_table.reshape(-1)
    adp_t = adaptive.transpose(0, 2, 1).reshape(_L * _ADIM, _N)

    mesh = plsc.VectorSubcoreMesh(core_axis_name="c", subcore_axis_name="s")
    fn = pl.kernel(
        _sc_body,
        out_type=jax.ShapeDtypeStruct((_NP, _OUT_D, _N), jnp.float32),
        mesh=mesh,
        compiler_params=pltpu.CompilerParams(use_tc_tiling_on_sc=True,
                                             needs_layout_passes=False),
        scratch_types=[
            pltpu.VMEM((_BLK * 3,), jnp.float32),
            pltpu.VMEM((_BLK * 3,), jnp.float32),
            pltpu.VMEM((_OUT_D, _BLK), jnp.float32),
            pltpu.VMEM((_OUT_D, _BLK), jnp.float32),
            pltpu.VMEM((_TS * _DIM,), jnp.float32),
            pltpu.VMEM((_DOW * _DIM,), jnp.float32),
            pltpu.VMEM((56,), jnp.float32),
            pltpu.SemaphoreType.DMA,
            pltpu.SemaphoreType.DMA,
            pltpu.SemaphoreType.DMA,
            pltpu.SemaphoreType.DMA,
            pltpu.SemaphoreType.DMA,
        ],
    )
    out = fn(inp_flat, ts1, dow1, wb, adp_t)
    out = out.reshape(_B, _L, _OUT_D, _N)
    return out.transpose(0, 1, 3, 2)


# speedup vs baseline: 1.0349x; 1.0349x over previous
import functools

import jax
import jax.numpy as jnp
from jax import lax
from jax.experimental import pallas as pl
from jax.experimental.pallas import tpu as pltpu
from jax.experimental.pallas import tpu_sc as plsc

_B, _L, _N, _C = 8, 12, 2048, 1
_DIM = 24
_ADIM = 80
_TS = 24 * 12
_DOW = 7
_TOT = _B * _L * _N
_OUT_D = 3 * _DIM + _ADIM
_NP = _B * _L
_NC = 2
_NS = 16
_NW = _NC * _NS
_TPW = _TOT // _NW
_BLK = 128
_NBLK = _TPW // _BLK


def _sc_body(inp_ref, ts_ref, dow_ref, wb_ref, adp_ref, out_ref,
             inp_v0, inp_v1, asm_v0, asm_v1, ts_v, dow_v, wb_v,
             si0, si1, sa, so0, so1):
    wid = lax.axis_index("s") * _NC + lax.axis_index("c")
    base0 = wid * _TPW
    pltpu.sync_copy(wb_ref, wb_v)
    pltpu.sync_copy(ts_ref, ts_v)
    pltpu.sync_copy(dow_ref, dow_v)

    lane = lax.iota(jnp.int32, 16)
    lane3 = lane * 3
    wds = [plsc.load_gather(wb_v, [jnp.full((16,), 1 + d, jnp.int32)])
           for d in range(_DIM)]
    bds = [plsc.load_gather(wb_v, [jnp.full((16,), 1 + _DIM + d, jnp.int32)])
           for d in range(_DIM)]

    bufs = ((inp_v0, asm_v0, si0, so0),
            (inp_v1, asm_v1, si1, so1))

    def pair_body(j, carry):
        for p in (0, 1):
            q = 1 - p
            inp_v, asm_v, s_in, s_out = bufs[p]
            inp_n, asm_n, s_in_n, s_out_n = bufs[q]
            k = 2 * j + p
            t0 = base0 + k * _BLK
            plane = t0 // _N
            n0 = lax.rem(t0, _N)

            def _drain_prev():
                pltpu.make_async_copy(
                    out_ref.at[0, :, pl.ds(0, _BLK)], asm_v, s_out).wait()

            pl.when(j >= 1)(_drain_prev)

            lrow = lax.rem(plane, _L) * _ADIM
            pltpu.async_copy(
                adp_ref.at[pl.ds(lrow, _ADIM), pl.ds(n0, _BLK)],
                asm_v.at[pl.ds(72, _ADIM), :], sa)

            pltpu.make_async_copy(
                inp_ref.at[pl.ds(0, _BLK * 3)], inp_v, s_in).wait()

            for g in range(_BLK // 16):
                base = g * 48
                feat = plsc.load_gather(inp_v, [lane3 + base])
                tsv = plsc.load_gather(inp_v, [lane3 + (base + 1)])
                dwv = plsc.load_gather(inp_v, [lane3 + (base + 2)])
                ts24 = tsv.astype(jnp.int32) * _DIM
                dw24 = dwv.astype(jnp.int32) * _DIM
                sl = pl.ds(g * 16, 16)
                for d in range(_DIM):
                    asm_v[d, sl] = feat * wds[d] + bds[d]
                for d in range(_DIM):
                    asm_v[24 + d, sl] = plsc.load_gather(ts_v, [ts24 + d])
                for d in range(_DIM):
                    asm_v[48 + d, sl] = plsc.load_gather(dow_v, [dw24 + d])

            nxt = base0 + lax.rem(k + 2, _NBLK) * _BLK
            pltpu.async_copy(inp_ref.at[pl.ds(nxt * 3, _BLK * 3)],
                             inp_v, s_in)

            pltpu.make_async_copy(
                adp_ref.at[pl.ds(0, _ADIM), pl.ds(0, _BLK)],
                asm_v.at[pl.ds(72, _ADIM), :], sa).wait()
            pltpu.async_copy(
                asm_v, out_ref.at[plane, :, pl.ds(n0, _BLK)], s_out)
        return carry

    pltpu.async_copy(inp_ref.at[pl.ds(base0 * 3, _BLK * 3)], inp_v0, si0)
    pltpu.async_copy(inp_ref.at[pl.ds((base0 + _BLK) * 3, _BLK * 3)],
                     inp_v1, si1)

    lax.fori_loop(0, _NBLK // 2, pair_body, 0)

    pltpu.make_async_copy(
        out_ref.at[0, :, pl.ds(0, _BLK)], asm_v0, so0).wait()
    pltpu.make_async_copy(
        out_ref.at[0, :, pl.ds(0, _BLK)], asm_v1, so1).wait()
    pltpu.make_async_copy(
        inp_ref.at[pl.ds(0, _BLK * 3)], inp_v0, si0).wait()
    pltpu.make_async_copy(
        inp_ref.at[pl.ds(0, _BLK * 3)], inp_v1, si1).wait()


@jax.jit
def kernel(input, W, b, ts_table, dow_table, adaptive):
    inp_flat = input.reshape(-1)
    wb = jnp.concatenate([jnp.zeros((1,), jnp.float32),
                          W.reshape(-1), b,
                          jnp.zeros((7,), jnp.float32)])
    ts1 = ts_table.reshape(-1)
    dow1 = dow_table.reshape(-1)
    adp_t = adaptive.transpose(0, 2, 1).reshape(_L * _ADIM, _N)

    mesh = plsc.VectorSubcoreMesh(core_axis_name="c", subcore_axis_name="s")
    fn = pl.kernel(
        _sc_body,
        out_type=jax.ShapeDtypeStruct((_NP, _OUT_D, _N), jnp.float32),
        mesh=mesh,
        compiler_params=pltpu.CompilerParams(use_tc_tiling_on_sc=True,
                                             needs_layout_passes=False),
        scratch_types=[
            pltpu.VMEM((_BLK * 3,), jnp.float32),
            pltpu.VMEM((_BLK * 3,), jnp.float32),
            pltpu.VMEM((_OUT_D, _BLK), jnp.float32),
            pltpu.VMEM((_OUT_D, _BLK), jnp.float32),
            pltpu.VMEM((_TS * _DIM,), jnp.float32),
            pltpu.VMEM((_DOW * _DIM,), jnp.float32),
            pltpu.VMEM((56,), jnp.float32),
            pltpu.SemaphoreType.DMA,
            pltpu.SemaphoreType.DMA,
            pltpu.SemaphoreType.DMA,
            pltpu.SemaphoreType.DMA,
            pltpu.SemaphoreType.DMA,
        ],
    )
    out = fn(inp_flat, ts1, dow1, wb, adp_t)
    out = out.reshape(_B, _L, _OUT_D, _N)
    return out.transpose(0, 1, 3, 2)
